# baseline (speedup 1.0000x reference)
import functools

import numpy as np

import jax
import jax.numpy as jnp
from jax import lax
from jax.experimental import pallas as pl
from jax.experimental.pallas import tpu as pltpu
from jax.experimental.pallas import tpu_sc as plsc

M = 2048
N = 4096
NV = N // 16
NB = 256
NC = 2
NS = 16
NW = NC * NS
RPW = M // NW
S = 4
NST = 2 * S
MINI32 = np.int32(-2147483648)


def _to_key(xi):
    mask = (xi >> 31) | MINI32
    return xi ^ mask


def _from_key(k):
    mask = ((~k) >> 31) | MINI32
    return lax.bitcast_convert_type(k ^ mask, jnp.float32)


def _digit(k, shift):
    ku = lax.bitcast_convert_type(k, jnp.uint32)
    return ((ku >> shift) & 255).astype(jnp.int32)


def _store_scalar(ref, idx, val, lane):
    idxv = jnp.broadcast_to(idx, (16,)).astype(jnp.int32)
    valv = jnp.broadcast_to(val, (16,))
    plsc.store_scatter(ref, [idxv], valv, mask=lane == 0)


def _sc_body(x_hbm, a_hbm, out_hbm, *scratch):
    k0 = scratch[0:NST]
    k1 = scratch[NST:2 * NST]
    cnt = scratch[2 * NST:3 * NST]
    resm, resw, resd = scratch[3 * NST:3 * NST + 3]

    wid = lax.axis_index("s") * NC + lax.axis_index("c")
    base = wid * RPW
    lane = lax.iota(jnp.int32, 16)
    stride_idx = lane * (NV + 1)
    zeros16 = jnp.zeros((16,), jnp.int32)
    ones16 = jnp.ones((16,), jnp.int32)

    def zero_cnt(unroll=8):
        def zero(i):
            for t in range(NST):
                cnt[t][pl.ds(i * 16, 16)] = zeros16
        plsc.parallel_loop(0, NB, unroll=unroll)(zero)

    def radix_pass(shift, srcs, dsts, first):
        if not first:
            zero_cnt()

            def hist(i):
                for t in range(NST):
                    kv = plsc.load_gather(srcs[t], [stride_idx + i])
                    d = _digit(kv, shift)
                    plsc.addupdate_scatter(cnt[t], [d * 16 + lane], ones16)
            lax.fori_loop(0, NV, lambda i, c: (hist(i), c)[1], 0,
                          unroll=4)

        def prefix(i, carry):
            newc = []
            for t in range(NST):
                v = cnt[t][pl.ds(i * 16, 16)]
                pcs = plsc.cumsum(v)
                cnt[t][pl.ds(i * 16, 16)] = pcs - v + carry[t]
                newc.append(carry[t] + jnp.sum(v))
            return tuple(newc)
        plsc.parallel_loop(0, NB, carry=(jnp.int32(0),) * NST,
                           unroll=2)(prefix)

        def perm(i, c):
            kv = [plsc.load_gather(srcs[t], [stride_idx + i])
                  for t in range(NST)]
            ci = [_digit(kv[t], shift) * 16 + lane for t in range(NST)]
            dest = [plsc.load_gather(cnt[t], [ci[t]]) for t in range(NST)]
            for t in range(NST):
                plsc.store_scatter(cnt[t], [ci[t]], dest[t] + 1)
                plsc.store_scatter(dsts[t], [dest[t] + (dest[t] >> 8)], kv[t])
            return c
        lax.fori_loop(0, NV, perm, 0, unroll=1)

    def row_body(r, c):
        for s in range(S):
            row = base + r * S + s
            pltpu.sync_copy(x_hbm.at[row], k1[2 * s].at[pl.ds(0, N)])
            pltpu.sync_copy(a_hbm.at[row], k1[2 * s + 1].at[pl.ds(0, N)])

        def conv(i):
            qb = i * 16 + (i >> 4)
            for t in range(NST):
                k = _to_key(k1[t][pl.ds(i * 16, 16)])
                k0[t][pl.ds(qb, 16)] = k
                d = _digit(k, 0)
                plsc.addupdate_scatter(cnt[t], [d * 16 + lane], ones16)
        lax.fori_loop(0, NV, lambda i, c: (conv(i), c)[1], 0, unroll=4)

        radix_pass(0, k0, k1, True)
        radix_pass(8, k1, k0, False)
        radix_pass(16, k0, k1, False)
        radix_pass(24, k1, k0, False)

        def wacc(i, carry):
            qb = i * 16 + (i >> 4)
            sums, diffs = carry
            f = [_from_key(k0[t][pl.ds(qb, 16)]) for t in range(NST)]
            nsums = tuple(sums[t] + f[t] for t in range(NST))
            ndiffs = tuple(diffs[s] + jnp.abs(f[2 * s] - f[2 * s + 1])
                           for s in range(S))
            for t in range(NST):
                cnt[t][pl.ds(i * 16, 16)] = zeros16
            return nsums, ndiffs
        zf = jnp.zeros((16,), jnp.float32)
        sums, diffs = plsc.parallel_loop(
            0, NV, carry=((zf,) * NST, (zf,) * S), unroll=4)(wacc)

        med_off = 2032 + (2032 >> 8)
        inv_n = np.float32(1.0 / N)
        for s in range(S):
            mx = _from_key(k0[2 * s][pl.ds(med_off, 16)])
            ma = _from_key(k0[2 * s + 1][pl.ds(med_off, 16)])
            med_d = jnp.sum(jnp.where(lane == 15, mx - ma, 0.0))
            sgn = jnp.sign(med_d)
            mean_d = (jnp.sum(sums[2 * s]) - jnp.sum(sums[2 * s + 1])) * inv_n
            idx = r * S + s
            _store_scalar(resm, idx, mean_d * sgn, lane)
            _store_scalar(resw, idx, jnp.sum(diffs[s]) * inv_n * sgn, lane)
            _store_scalar(resd, idx, med_d, lane)
        return c

    zero_cnt(unroll=8)
    lax.fori_loop(0, RPW // S, row_body, 0)

    pltpu.sync_copy(resm, out_hbm.at[0, pl.ds(base, RPW)])
    pltpu.sync_copy(resw, out_hbm.at[1, pl.ds(base, RPW)])
    pltpu.sync_copy(resd, out_hbm.at[2, pl.ds(base, RPW)])


@functools.lru_cache(maxsize=None)
def _build():
    scratch = (
        [pltpu.VMEM((N + 16,), jnp.int32) for _ in range(NST)]
        + [pltpu.VMEM((N + 16,), jnp.int32) for _ in range(NST)]
        + [pltpu.VMEM((NB * 16,), jnp.int32) for _ in range(NST)]
        + [pltpu.VMEM((RPW,), jnp.float32) for _ in range(3)]
    )
    return pl.kernel(
        _sc_body,
        out_type=jax.ShapeDtypeStruct((3, M), jnp.float32),
        mesh=plsc.VectorSubcoreMesh(core_axis_name="c", subcore_axis_name="s"),
        compiler_params=pltpu.CompilerParams(needs_layout_passes=False),
        scratch_types=scratch,
    )


def kernel(x, anchor_features):
    xi = lax.bitcast_convert_type(x, jnp.int32)
    ai = lax.bitcast_convert_type(anchor_features, jnp.int32)
    return _build()(xi, ai)


# speedup vs baseline: 1.9959x; 1.9959x over previous
import functools

import numpy as np

import jax
import jax.numpy as jnp
from jax import lax
from jax.experimental import pallas as pl
from jax.experimental.pallas import tpu as pltpu
from jax.experimental.pallas import tpu_sc as plsc

M = 2048
N = 4096
NV = N // 16
NB = 256
NC = 2
NS = 16
NW = NC * NS
RPW = M // NW
S = 4
NST = 2 * S
MINI32 = np.int32(-2147483648)


def _to_key(xi):
    mask = (xi >> 31) | MINI32
    return xi ^ mask


def _from_key(k):
    mask = ((~k) >> 31) | MINI32
    return lax.bitcast_convert_type(k ^ mask, jnp.float32)


def _digit(k, shift):
    ku = lax.bitcast_convert_type(k, jnp.uint32)
    return ((ku >> shift) & 255).astype(jnp.int32)


def _store_scalar(ref, idx, val, lane):
    idxv = jnp.broadcast_to(idx, (16,)).astype(jnp.int32)
    valv = jnp.broadcast_to(val, (16,))
    plsc.store_scatter(ref, [idxv], valv, mask=lane == 0)


def _sc_body(x_hbm, a_hbm, out_hbm, *scratch):
    k0 = scratch[0:NST]
    k1 = scratch[NST:2 * NST]
    cnt = scratch[2 * NST:3 * NST]
    resm, resw, resd = scratch[3 * NST:3 * NST + 3]

    wid = lax.axis_index("s") * NC + lax.axis_index("c")
    base = wid * RPW
    lane = lax.iota(jnp.int32, 16)
    stride_idx = lane * (NV + 1)
    zeros16 = jnp.zeros((16,), jnp.int32)
    ones16 = jnp.ones((16,), jnp.int32)

    def zero_cnt(unroll=4):
        def zero(i):
            for t in range(NST):
                cnt[t][pl.ds(i * 16, 16)] = zeros16
        plsc.parallel_loop(0, NB, unroll=unroll)(zero)

    def radix_pass(shift, srcs, dsts, first):
        if not first:
            zero_cnt()

            def hist(i):
                for t in range(NST):
                    kv = plsc.load_gather(srcs[t], [stride_idx + i])
                    d = _digit(kv, shift)
                    plsc.addupdate_scatter(cnt[t], [d * 16 + lane], ones16)
            plsc.parallel_loop(0, NV, unroll=2)(hist)

        def prefix(i, carry):
            newc = []
            for t in range(NST):
                v = cnt[t][pl.ds(i * 16, 16)]
                pcs = plsc.cumsum(v)
                cnt[t][pl.ds(i * 16, 16)] = pcs - v + carry[t]
                newc.append(carry[t] + jnp.sum(v))
            return tuple(newc)
        plsc.parallel_loop(0, NB, carry=(jnp.int32(0),) * NST,
                           unroll=2)(prefix)

        def perm(i, c):
            kv = [plsc.load_gather(srcs[t], [stride_idx + i])
                  for t in range(NST)]
            ci = [_digit(kv[t], shift) * 16 + lane for t in range(NST)]
            dest = [plsc.load_gather(cnt[t], [ci[t]]) for t in range(NST)]
            for t in range(NST):
                plsc.store_scatter(cnt[t], [ci[t]], dest[t] + 1)
                plsc.store_scatter(dsts[t], [dest[t] + (dest[t] >> 8)], kv[t])
            return c
        lax.fori_loop(0, NV, perm, 0, unroll=1)

    def row_body(r, c):
        for s in range(S):
            row = base + r * S + s
            pltpu.sync_copy(x_hbm.at[row], k1[2 * s].at[pl.ds(0, N)])
            pltpu.sync_copy(a_hbm.at[row], k1[2 * s + 1].at[pl.ds(0, N)])

        def conv(i):
            qb = i * 16 + (i >> 4)
            for t in range(NST):
                k = _to_key(k1[t][pl.ds(i * 16, 16)])
                k0[t][pl.ds(qb, 16)] = k
                d = _digit(k, 0)
                plsc.addupdate_scatter(cnt[t], [d * 16 + lane], ones16)
        plsc.parallel_loop(0, NV, unroll=2)(conv)

        radix_pass(0, k0, k1, True)
        radix_pass(8, k1, k0, False)
        radix_pass(16, k0, k1, False)
        radix_pass(24, k1, k0, False)

        def wacc(i, carry):
            qb = i * 16 + (i >> 4)
            sums, diffs = carry
            f = [_from_key(k0[t][pl.ds(qb, 16)]) for t in range(NST)]
            nsums = tuple(sums[t] + f[t] for t in range(NST))
            ndiffs = tuple(diffs[s] + jnp.abs(f[2 * s] - f[2 * s + 1])
                           for s in range(S))
            for t in range(NST):
                cnt[t][pl.ds(i * 16, 16)] = zeros16
            return nsums, ndiffs
        zf = jnp.zeros((16,), jnp.float32)
        sums, diffs = plsc.parallel_loop(
            0, NV, carry=((zf,) * NST, (zf,) * S), unroll=2)(wacc)

        med_off = 2032 + (2032 >> 8)
        inv_n = np.float32(1.0 / N)
        for s in range(S):
            mx = _from_key(k0[2 * s][pl.ds(med_off, 16)])
            ma = _from_key(k0[2 * s + 1][pl.ds(med_off, 16)])
            med_d = jnp.sum(jnp.where(lane == 15, mx - ma, 0.0))
            sgn = jnp.sign(med_d)
            mean_d = (jnp.sum(sums[2 * s]) - jnp.sum(sums[2 * s + 1])) * inv_n
            idx = r * S + s
            _store_scalar(resm, idx, mean_d * sgn, lane)
            _store_scalar(resw, idx, jnp.sum(diffs[s]) * inv_n * sgn, lane)
            _store_scalar(resd, idx, med_d, lane)
        return c

    zero_cnt(unroll=8)
    lax.fori_loop(0, RPW // S, row_body, 0)

    pltpu.sync_copy(resm, out_hbm.at[0, pl.ds(base, RPW)])
    pltpu.sync_copy(resw, out_hbm.at[1, pl.ds(base, RPW)])
    pltpu.sync_copy(resd, out_hbm.at[2, pl.ds(base, RPW)])


@functools.lru_cache(maxsize=None)
def _build():
    scratch = (
        [pltpu.VMEM((N + 16,), jnp.int32) for _ in range(NST)]
        + [pltpu.VMEM((N + 16,), jnp.int32) for _ in range(NST)]
        + [pltpu.VMEM((NB * 16,), jnp.int32) for _ in range(NST)]
        + [pltpu.VMEM((RPW,), jnp.float32) for _ in range(3)]
    )
    return pl.kernel(
        _sc_body,
        out_type=jax.ShapeDtypeStruct((3, M), jnp.float32),
        mesh=plsc.VectorSubcoreMesh(core_axis_name="c", subcore_axis_name="s"),
        compiler_params=pltpu.CompilerParams(needs_layout_passes=False),
        scratch_types=scratch,
    )


def kernel(x, anchor_features):
    xi = lax.bitcast_convert_type(x, jnp.int32)
    ai = lax.bitcast_convert_type(anchor_features, jnp.int32)
    return _build()(xi, ai)


# speedup vs baseline: 2.2284x; 1.1165x over previous
import functools

import numpy as np

import jax
import jax.numpy as jnp
from jax import lax
from jax.experimental import pallas as pl
from jax.experimental.pallas import tpu as pltpu
from jax.experimental.pallas import tpu_sc as plsc

M = 2048
N = 4096
NV = N // 16
NB = 256
NC = 2
NS = 16
NW = NC * NS
RPW = M // NW
S = 4
NST = 2 * S
MINI32 = np.int32(-2147483648)


def _to_key(xi):
    mask = (xi >> 31) | MINI32
    return xi ^ mask


def _from_key(k):
    mask = ((~k) >> 31) | MINI32
    return lax.bitcast_convert_type(k ^ mask, jnp.float32)


def _digit(k, shift):
    ku = lax.bitcast_convert_type(k, jnp.uint32)
    return ((ku >> shift) & 255).astype(jnp.int32)


def _store_scalar(ref, idx, val, lane):
    idxv = jnp.broadcast_to(idx, (16,)).astype(jnp.int32)
    valv = jnp.broadcast_to(val, (16,))
    plsc.store_scatter(ref, [idxv], valv, mask=lane == 0)


def _sc_body(x_hbm, a_hbm, out_hbm, *scratch):
    k0 = scratch[0:NST]
    k1 = scratch[NST:2 * NST]
    cnt = scratch[2 * NST:3 * NST]
    resm, resw, resd = scratch[3 * NST:3 * NST + 3]
    dma_sem = scratch[3 * NST + 3]

    wid = lax.axis_index("s") * NC + lax.axis_index("c")
    base = wid * RPW
    lane = lax.iota(jnp.int32, 16)
    stride_idx = lane * (NV + 1)
    zeros16 = jnp.zeros((16,), jnp.int32)
    ones16 = jnp.ones((16,), jnp.int32)

    def zero_cnt(unroll=4):
        def zero(i):
            for t in range(NST):
                cnt[t][pl.ds(i * 16, 16)] = zeros16
        plsc.parallel_loop(0, NB, unroll=unroll)(zero)

    def radix_pass(shift, srcs, dsts, first):
        if not first:
            zero_cnt()

            def hist(i):
                for t in range(NST):
                    kv = plsc.load_gather(srcs[t], [stride_idx + i])
                    d = _digit(kv, shift)
                    plsc.addupdate_scatter(cnt[t], [d * 16 + lane], ones16)
            plsc.parallel_loop(0, NV, unroll=2)(hist)

        def prefix(i, carry):
            newc = []
            for t in range(NST):
                v = cnt[t][pl.ds(i * 16, 16)]
                pcs = plsc.cumsum(v)
                cnt[t][pl.ds(i * 16, 16)] = pcs - v + carry[t]
                newc.append(carry[t] + jnp.sum(v))
            return tuple(newc)
        plsc.parallel_loop(0, NB, carry=(jnp.int32(0),) * NST,
                           unroll=2)(prefix)

        def perm(i, c):
            kv = [plsc.load_gather(srcs[t], [stride_idx + i])
                  for t in range(NST)]
            ci = [_digit(kv[t], shift) * 16 + lane for t in range(NST)]
            dest = [plsc.load_gather(cnt[t], [ci[t]]) for t in range(NST)]
            for t in range(NST):
                plsc.store_scatter(cnt[t], [ci[t]], dest[t] + 1)
                plsc.store_scatter(dsts[t], [dest[t] + (dest[t] >> 8)], kv[t])
            return c
        lax.fori_loop(0, NV, perm, 0, unroll=1)

    def issue_group(r):
        for s in range(S):
            row = jnp.minimum(base + r * S + s, M - 1)
            pltpu.async_copy(x_hbm.at[row], k1[2 * s].at[pl.ds(0, N)],
                             dma_sem)
            pltpu.async_copy(a_hbm.at[row], k1[2 * s + 1].at[pl.ds(0, N)],
                             dma_sem)

    def wait_group(r):
        for s in range(S):
            row = jnp.minimum(base + r * S + s, M - 1)
            pltpu.make_async_copy(x_hbm.at[row], k1[2 * s].at[pl.ds(0, N)],
                                  dma_sem).wait()
            pltpu.make_async_copy(a_hbm.at[row],
                                  k1[2 * s + 1].at[pl.ds(0, N)],
                                  dma_sem).wait()

    def row_body(r, c):
        wait_group(r)

        def conv(i):
            qb = i * 16 + (i >> 4)
            for t in range(NST):
                k = _to_key(k1[t][pl.ds(i * 16, 16)])
                k0[t][pl.ds(qb, 16)] = k
                d = _digit(k, 0)
                plsc.addupdate_scatter(cnt[t], [d * 16 + lane], ones16)
        plsc.parallel_loop(0, NV, unroll=2)(conv)

        radix_pass(0, k0, k1, True)
        radix_pass(8, k1, k0, False)
        radix_pass(16, k0, k1, False)
        radix_pass(24, k1, k0, False)
        issue_group(r + 1)

        def wacc(i, carry):
            qb = i * 16 + (i >> 4)
            sums, diffs = carry
            f = [_from_key(k0[t][pl.ds(qb, 16)]) for t in range(NST)]
            nsums = tuple(sums[t] + f[t] for t in range(NST))
            ndiffs = tuple(diffs[s] + jnp.abs(f[2 * s] - f[2 * s + 1])
                           for s in range(S))
            for t in range(NST):
                cnt[t][pl.ds(i * 16, 16)] = zeros16
            return nsums, ndiffs
        zf = jnp.zeros((16,), jnp.float32)
        sums, diffs = plsc.parallel_loop(
            0, NV, carry=((zf,) * NST, (zf,) * S), unroll=2)(wacc)

        med_off = 2032 + (2032 >> 8)
        inv_n = np.float32(1.0 / N)
        for s in range(S):
            mx = _from_key(k0[2 * s][pl.ds(med_off, 16)])
            ma = _from_key(k0[2 * s + 1][pl.ds(med_off, 16)])
            med_d = jnp.sum(jnp.where(lane == 15, mx - ma, 0.0))
            sgn = jnp.sign(med_d)
            mean_d = (jnp.sum(sums[2 * s]) - jnp.sum(sums[2 * s + 1])) * inv_n
            idx = r * S + s
            _store_scalar(resm, idx, mean_d * sgn, lane)
            _store_scalar(resw, idx, jnp.sum(diffs[s]) * inv_n * sgn, lane)
            _store_scalar(resd, idx, med_d, lane)
        return c

    issue_group(0)
    zero_cnt(unroll=8)
    lax.fori_loop(0, RPW // S, row_body, 0)
    wait_group(RPW // S)

    pltpu.sync_copy(resm, out_hbm.at[0, pl.ds(base, RPW)])
    pltpu.sync_copy(resw, out_hbm.at[1, pl.ds(base, RPW)])
    pltpu.sync_copy(resd, out_hbm.at[2, pl.ds(base, RPW)])


@functools.lru_cache(maxsize=None)
def _build():
    scratch = (
        [pltpu.VMEM((N + 16,), jnp.int32) for _ in range(NST)]
        + [pltpu.VMEM((N + 16,), jnp.int32) for _ in range(NST)]
        + [pltpu.VMEM((NB * 16,), jnp.int32) for _ in range(NST)]
        + [pltpu.VMEM((RPW,), jnp.float32) for _ in range(3)]
        + [pltpu.SemaphoreType.DMA]
    )
    return pl.kernel(
        _sc_body,
        out_type=jax.ShapeDtypeStruct((3, M), jnp.float32),
        mesh=plsc.VectorSubcoreMesh(core_axis_name="c", subcore_axis_name="s"),
        compiler_params=pltpu.CompilerParams(needs_layout_passes=False),
        scratch_types=scratch,
    )


def kernel(x, anchor_features):
    xi = lax.bitcast_convert_type(x, jnp.int32)
    ai = lax.bitcast_convert_type(anchor_features, jnp.int32)
    return _build()(xi, ai)
